# baseline (speedup 1.0000x reference)
import functools

import jax
import jax.numpy as jnp
from jax import lax
from jax.experimental import pallas as pl
from jax.experimental.pallas import tpu as pltpu
from jax.experimental.pallas import tpu_sc as plsc

N = 10000
E = 640000
D_IN = 8710
D_HID = 128
D_OUT = 70
D_PAD = 128

NC = 2
NS = 16
NW = NC * NS
CHUNK = 128
K4 = 2
EPT = -(-E // NW // (CHUNK * K4)) * CHUNK * K4
EPAD = EPT * NW
NCH4 = EPT // (CHUNK * K4)
NPAD = 10240
RPT = NPAD // NS
JUNK = N + 8

BM = 400
NBLK = N // BM

_f32 = jnp.float32


@functools.cache
def _deg_kernel_fn():
    mesh = plsc.VectorSubcoreMesh(core_axis_name="c", subcore_axis_name="s")

    @functools.partial(
        pl.kernel, mesh=mesh,
        out_type=jax.ShapeDtypeStruct((NC, NPAD, 128), _f32),
        scratch_types=[
            pltpu.VMEM((K4, 2, CHUNK), jnp.int32),
            pltpu.VMEM((CHUNK, 128), _f32),
            pltpu.VMEM_SHARED((NPAD, 128), _f32),
        ],
    )
    def deg_kernel(edges_hbm, zeros_hbm, ones_hbm, out_hbm,
                   e_all, ones_v, acc_sh):
        c = lax.axis_index("c")
        s = lax.axis_index("s")
        g = c * NS + s
        r0 = s * RPT
        pltpu.sync_copy(ones_hbm, ones_v)
        pltpu.sync_copy(zeros_hbm.at[pl.ds(r0, RPT)], acc_sh.at[pl.ds(r0, RPT)])
        plsc.subcore_barrier()

        def body(jj, carry):
            blk = g * NCH4 + jj
            pltpu.sync_copy(edges_hbm.at[blk], e_all)
            for k in range(K4):
                pltpu.sync_copy(ones_v, acc_sh.at[e_all.at[k, 1]], add=True)
            return carry

        lax.fori_loop(0, NCH4, body, 0)
        plsc.subcore_barrier()
        pltpu.sync_copy(acc_sh.at[pl.ds(r0, RPT)], out_hbm.at[c, pl.ds(r0, RPT)])

    return deg_kernel


@functools.cache
def _make_prop(D):
    mesh = plsc.VectorSubcoreMesh(core_axis_name="c", subcore_axis_name="s")

    @functools.partial(
        pl.kernel, mesh=mesh,
        out_type=jax.ShapeDtypeStruct((NC, NPAD, D), _f32),
        scratch_types=[
            pltpu.VMEM((K4, 2, CHUNK), jnp.int32),
            pltpu.VMEM((K4, CHUNK, D), _f32),
            pltpu.VMEM_SHARED((NPAD, D), _f32),
        ] + [pltpu.SemaphoreType.DMA] * K4,
    )
    def prop(edges_hbm, table_hbm, zeros_hbm, out_hbm,
             e_all, rows, acc_sh, *gsem):
        c = lax.axis_index("c")
        s = lax.axis_index("s")
        g = c * NS + s
        r0 = s * RPT
        pltpu.sync_copy(zeros_hbm.at[pl.ds(r0, RPT)], acc_sh.at[pl.ds(r0, RPT)])
        plsc.subcore_barrier()

        def body(jj, carry):
            blk = g * NCH4 + jj
            pltpu.sync_copy(edges_hbm.at[blk], e_all)
            gcs = [
                pltpu.async_copy(table_hbm.at[e_all.at[k, 0]], rows.at[k],
                                 gsem[k])
                for k in range(K4)
            ]
            for k in range(K4):
                gcs[k].wait()
                pltpu.sync_copy(rows.at[k], acc_sh.at[e_all.at[k, 1]],
                                add=True)
            return carry

        lax.fori_loop(0, NCH4, body, 0)
        plsc.subcore_barrier()
        pltpu.sync_copy(acc_sh.at[pl.ds(r0, RPT)], out_hbm.at[c, pl.ds(r0, RPT)])

    return prop


def _dinv_from(deg_ref):
    dv = deg_ref[...]
    return lax.rsqrt(dv[0, :, :1] + dv[1, :, :1] + 1.0)


def _mm1_body(x_ref, w_ref, out_ref):
    out_ref[...] = jnp.dot(x_ref[...], w_ref[...],
                           preferred_element_type=_f32)


def _scale_body(h_ref, deg_ref, out_ref):
    out_ref[...] = h_ref[...] * _dinv_from(deg_ref)


def _fuse2_body(acc_ref, t1_ref, deg_ref, b1_ref, w2_ref, out_ref):
    dinv = _dinv_from(deg_ref)
    a = acc_ref[...]
    z = (a[0] + a[1] + t1_ref[...]) * dinv + b1_ref[...]
    z = jnp.maximum(z, 0.0)
    out_ref[...] = jnp.dot(z, w2_ref[...], preferred_element_type=_f32) * dinv


def _fuse3_body(acc_ref, t2_ref, deg_ref, b2_ref, out_ref):
    dinv = _dinv_from(deg_ref)
    a = acc_ref[...]
    z = (a[0] + a[1] + t2_ref[...]) * dinv + b2_ref[...]
    sm = z[:, :D_OUT]
    m = jnp.max(sm, axis=1, keepdims=True)
    e = jnp.exp(sm - m)
    out_ref[...] = e / jnp.sum(e, axis=1, keepdims=True)


def _mm1(x, w1):
    return pl.pallas_call(
        _mm1_body,
        grid=(NBLK,),
        in_specs=[
            pl.BlockSpec((BM, D_IN), lambda i: (i, 0)),
            pl.BlockSpec((D_IN, D_HID), lambda i: (0, 0)),
        ],
        out_specs=pl.BlockSpec((BM, D_HID), lambda i: (i, 0)),
        out_shape=jax.ShapeDtypeStruct((N, D_HID), _f32),
    )(x, w1)


def _scale(h1, deg_raw):
    return pl.pallas_call(
        _scale_body,
        grid=(NBLK,),
        in_specs=[
            pl.BlockSpec((BM, D_HID), lambda i: (i, 0)),
            pl.BlockSpec((NC, BM, 128), lambda i: (0, i, 0)),
        ],
        out_specs=pl.BlockSpec((BM, D_HID), lambda i: (i, 0)),
        out_shape=jax.ShapeDtypeStruct((N, D_HID), _f32),
    )(h1, deg_raw)


def _fuse2(agg1, t1, deg_raw, b1, w2pad):
    return pl.pallas_call(
        _fuse2_body,
        grid=(NBLK,),
        in_specs=[
            pl.BlockSpec((NC, BM, D_HID), lambda i: (0, i, 0)),
            pl.BlockSpec((BM, D_HID), lambda i: (i, 0)),
            pl.BlockSpec((NC, BM, 128), lambda i: (0, i, 0)),
            pl.BlockSpec((1, D_HID), lambda i: (0, 0)),
            pl.BlockSpec((D_HID, D_PAD), lambda i: (0, 0)),
        ],
        out_specs=pl.BlockSpec((BM, D_PAD), lambda i: (i, 0)),
        out_shape=jax.ShapeDtypeStruct((N, D_PAD), _f32),
    )(agg1, t1, deg_raw, b1, w2pad)


def _fuse3(agg2, t2, deg_raw, b2):
    return pl.pallas_call(
        _fuse3_body,
        grid=(NBLK,),
        in_specs=[
            pl.BlockSpec((NC, BM, D_PAD), lambda i: (0, i, 0)),
            pl.BlockSpec((BM, D_PAD), lambda i: (i, 0)),
            pl.BlockSpec((NC, BM, 128), lambda i: (0, i, 0)),
            pl.BlockSpec((1, D_PAD), lambda i: (0, 0)),
        ],
        out_specs=pl.BlockSpec((BM, D_OUT), lambda i: (i, 0)),
        out_shape=jax.ShapeDtypeStruct((N, D_OUT), _f32),
    )(agg2, t2, deg_raw, b2)


def kernel(inputs, edges, W1, b1, W2, b2):
    src = edges[0]
    dst = edges[1]
    ept_real = E // NW
    padcols = EPT - ept_real
    src_pad = jnp.pad(src.reshape(NW, ept_real),
                      ((0, 0), (0, padcols))).reshape(-1)
    jpt = (NPAD - N) // NW
    junk = (N + jnp.arange(NW, dtype=jnp.int32)[:, None] * jpt
            + jnp.arange(padcols, dtype=jnp.int32)[None, :] % jpt)
    dst_pad = jnp.concatenate([dst.reshape(NW, ept_real), junk],
                              axis=1).reshape(-1)
    e_arr = jnp.stack(
        [src_pad.reshape(-1, K4, CHUNK), dst_pad.reshape(-1, K4, CHUNK)],
        axis=2)

    zeros128 = jnp.zeros((NPAD, 128), _f32)
    ones128 = jnp.ones((CHUNK, 128), _f32)
    w2pad = jnp.pad(W2, ((0, 0), (0, D_PAD - D_OUT)))
    b2pad = jnp.pad(b2, (0, D_PAD - D_OUT)).reshape(1, D_PAD)

    deg_raw = _deg_kernel_fn()(e_arr, zeros128, ones128)
    h1 = _mm1(inputs, W1)
    t1 = _scale(h1, deg_raw)
    agg1 = _make_prop(D_HID)(e_arr, t1, zeros128)
    t2 = _fuse2(agg1, t1, deg_raw, b1.reshape(1, D_HID), w2pad)
    agg2 = _make_prop(D_PAD)(e_arr, t2, zeros128)
    return _fuse3(agg2, t2, deg_raw, b2pad)


# speedup vs baseline: 1.0110x; 1.0110x over previous
import functools

import jax
import jax.numpy as jnp
from jax import lax
from jax.experimental import pallas as pl
from jax.experimental.pallas import tpu as pltpu
from jax.experimental.pallas import tpu_sc as plsc

N = 10000
E = 640000
D_IN = 8710
D_HID = 128
D_OUT = 70
D_PAD = 128

NC = 2
NS = 16
NW = NC * NS
CHUNK = 128
K4 = 2
EPT = -(-E // NW // (CHUNK * K4)) * CHUNK * K4
EPAD = EPT * NW
NCH4 = EPT // (CHUNK * K4)
NPAD = 10240
RPT = NPAD // NS
JUNK = N + 8

BM = 400
NBLK = N // BM

_f32 = jnp.float32


@functools.cache
def _deg_kernel_fn():
    mesh = plsc.VectorSubcoreMesh(core_axis_name="c", subcore_axis_name="s")

    @functools.partial(
        pl.kernel, mesh=mesh,
        out_type=jax.ShapeDtypeStruct((NC, NPAD, 128), _f32),
        scratch_types=[
            pltpu.VMEM((K4, 2, CHUNK), jnp.int32),
            pltpu.VMEM((CHUNK, 128), _f32),
            pltpu.VMEM_SHARED((NPAD, 128), _f32),
        ],
    )
    def deg_kernel(edges_hbm, zeros_hbm, ones_hbm, out_hbm,
                   e_all, ones_v, acc_sh):
        c = lax.axis_index("c")
        s = lax.axis_index("s")
        g = c * NS + s
        r0 = s * RPT
        pltpu.sync_copy(ones_hbm, ones_v)
        pltpu.sync_copy(zeros_hbm.at[pl.ds(r0, RPT)], acc_sh.at[pl.ds(r0, RPT)])
        plsc.subcore_barrier()

        def body(jj, carry):
            blk = g * NCH4 + jj
            pltpu.sync_copy(edges_hbm.at[blk], e_all)
            for k in range(K4):
                pltpu.sync_copy(ones_v, acc_sh.at[e_all.at[k, 1]], add=True)
            return carry

        lax.fori_loop(0, NCH4, body, 0)
        plsc.subcore_barrier()
        pltpu.sync_copy(acc_sh.at[pl.ds(r0, RPT)], out_hbm.at[c, pl.ds(r0, RPT)])

    return deg_kernel


@functools.cache
def _make_prop(D):
    mesh = plsc.VectorSubcoreMesh(core_axis_name="c", subcore_axis_name="s")

    @functools.partial(
        pl.kernel, mesh=mesh,
        out_type=jax.ShapeDtypeStruct((NC, NPAD, D), _f32),
        scratch_types=[
            pltpu.VMEM((K4, 2, CHUNK), jnp.int32),
            pltpu.VMEM((K4, CHUNK, D), _f32),
            pltpu.VMEM_SHARED((NPAD, D), _f32),
        ] + [pltpu.SemaphoreType.DMA] * K4,
    )
    def prop(edges_hbm, table_hbm, zeros_hbm, out_hbm,
             e_all, rows, acc_sh, *gsem):
        c = lax.axis_index("c")
        s = lax.axis_index("s")
        g = c * NS + s
        r0 = s * RPT
        pltpu.sync_copy(zeros_hbm.at[pl.ds(r0, RPT)], acc_sh.at[pl.ds(r0, RPT)])
        plsc.subcore_barrier()

        def body(jj, carry):
            blk = g * NCH4 + jj
            pltpu.sync_copy(edges_hbm.at[blk], e_all)
            gcs = [
                pltpu.async_copy(table_hbm.at[e_all.at[k, 0]], rows.at[k],
                                 gsem[k])
                for k in range(K4)
            ]
            for k in range(K4):
                gcs[k].wait()
                pltpu.sync_copy(rows.at[k], acc_sh.at[e_all.at[k, 1]],
                                add=True)
            return carry

        lax.fori_loop(0, NCH4, body, 0)
        plsc.subcore_barrier()
        pltpu.sync_copy(acc_sh.at[pl.ds(r0, RPT)], out_hbm.at[c, pl.ds(r0, RPT)])

    return prop


def _dinv_from(deg_ref):
    dv = deg_ref[...]
    return lax.rsqrt(dv[0, :, :1] + dv[1, :, :1] + 1.0)


def _mm1_body(x_ref, w_ref, deg_ref, out_ref):
    out_ref[...] = jnp.dot(x_ref[...], w_ref[...],
                           preferred_element_type=_f32) * _dinv_from(deg_ref)


def _fuse2_body(acc_ref, t1_ref, deg_ref, b1_ref, w2_ref, out_ref):
    dinv = _dinv_from(deg_ref)
    a = acc_ref[...]
    z = (a[0] + a[1] + t1_ref[...]) * dinv + b1_ref[...]
    z = jnp.maximum(z, 0.0)
    out_ref[...] = jnp.dot(z, w2_ref[...], preferred_element_type=_f32) * dinv


def _fuse3_body(acc_ref, t2_ref, deg_ref, b2_ref, out_ref):
    dinv = _dinv_from(deg_ref)
    a = acc_ref[...]
    z = (a[0] + a[1] + t2_ref[...]) * dinv + b2_ref[...]
    sm = z[:, :D_OUT]
    m = jnp.max(sm, axis=1, keepdims=True)
    e = jnp.exp(sm - m)
    out_ref[...] = e / jnp.sum(e, axis=1, keepdims=True)


def _mm1(x, w1, deg_raw):
    return pl.pallas_call(
        _mm1_body,
        grid=(NBLK,),
        in_specs=[
            pl.BlockSpec((BM, D_IN), lambda i: (i, 0)),
            pl.BlockSpec((D_IN, D_HID), lambda i: (0, 0)),
            pl.BlockSpec((NC, BM, 128), lambda i: (0, i, 0)),
        ],
        out_specs=pl.BlockSpec((BM, D_HID), lambda i: (i, 0)),
        out_shape=jax.ShapeDtypeStruct((N, D_HID), _f32),
    )(x, w1, deg_raw)


def _fuse2(agg1, t1, deg_raw, b1, w2pad):
    return pl.pallas_call(
        _fuse2_body,
        grid=(NBLK,),
        in_specs=[
            pl.BlockSpec((NC, BM, D_HID), lambda i: (0, i, 0)),
            pl.BlockSpec((BM, D_HID), lambda i: (i, 0)),
            pl.BlockSpec((NC, BM, 128), lambda i: (0, i, 0)),
            pl.BlockSpec((1, D_HID), lambda i: (0, 0)),
            pl.BlockSpec((D_HID, D_PAD), lambda i: (0, 0)),
        ],
        out_specs=pl.BlockSpec((BM, D_PAD), lambda i: (i, 0)),
        out_shape=jax.ShapeDtypeStruct((N, D_PAD), _f32),
    )(agg1, t1, deg_raw, b1, w2pad)


def _fuse3(agg2, t2, deg_raw, b2):
    return pl.pallas_call(
        _fuse3_body,
        grid=(NBLK,),
        in_specs=[
            pl.BlockSpec((NC, BM, D_PAD), lambda i: (0, i, 0)),
            pl.BlockSpec((BM, D_PAD), lambda i: (i, 0)),
            pl.BlockSpec((NC, BM, 128), lambda i: (0, i, 0)),
            pl.BlockSpec((1, D_PAD), lambda i: (0, 0)),
        ],
        out_specs=pl.BlockSpec((BM, D_OUT), lambda i: (i, 0)),
        out_shape=jax.ShapeDtypeStruct((N, D_OUT), _f32),
    )(agg2, t2, deg_raw, b2)


def kernel(inputs, edges, W1, b1, W2, b2):
    src = edges[0]
    dst = edges[1]
    ept_real = E // NW
    padcols = EPT - ept_real
    src_pad = jnp.pad(src.reshape(NW, ept_real),
                      ((0, 0), (0, padcols))).reshape(-1)
    jpt = (NPAD - N) // NW
    junk = (N + jnp.arange(NW, dtype=jnp.int32)[:, None] * jpt
            + jnp.arange(padcols, dtype=jnp.int32)[None, :] % jpt)
    dst_pad = jnp.concatenate([dst.reshape(NW, ept_real), junk],
                              axis=1).reshape(-1)
    e_arr = jnp.stack(
        [src_pad.reshape(-1, K4, CHUNK), dst_pad.reshape(-1, K4, CHUNK)],
        axis=2)

    zeros128 = jnp.zeros((NPAD, 128), _f32)
    ones128 = jnp.ones((CHUNK, 128), _f32)
    w2pad = jnp.pad(W2, ((0, 0), (0, D_PAD - D_OUT)))
    b2pad = jnp.pad(b2, (0, D_PAD - D_OUT)).reshape(1, D_PAD)

    deg_raw = _deg_kernel_fn()(e_arr, zeros128, ones128)
    t1 = _mm1(inputs, W1, deg_raw)
    agg1 = _make_prop(D_HID)(e_arr, t1, zeros128)
    t2 = _fuse2(agg1, t1, deg_raw, b1.reshape(1, D_HID), w2pad)
    agg2 = _make_prop(D_PAD)(e_arr, t2, zeros128)
    return _fuse3(agg2, t2, deg_raw, b2pad)


# speedup vs baseline: 1.0438x; 1.0324x over previous
import functools

import jax
import jax.numpy as jnp
from jax import lax
from jax.experimental import pallas as pl
from jax.experimental.pallas import tpu as pltpu
from jax.experimental.pallas import tpu_sc as plsc

N = 10000
E = 640000
D_IN = 8710
D_HID = 128
D_OUT = 70
D_PAD = 128

NC = 2
NS = 16
NW = NC * NS
CHUNK = 128
EPT = -(-E // NW // CHUNK) * CHUNK
EPAD = EPT * NW
NCH = EPT // CHUNK
NPAD = 10240
RPT = NPAD // NS
JUNK = N + 8

BM = 400
NBLK = N // BM

_f32 = jnp.float32


@functools.cache
def _deg_kernel_fn():
    mesh = plsc.VectorSubcoreMesh(core_axis_name="c", subcore_axis_name="s")

    @functools.partial(
        pl.kernel, mesh=mesh,
        out_type=jax.ShapeDtypeStruct((NC, NPAD, 128), _f32),
        scratch_types=[
            pltpu.VMEM((CHUNK,), jnp.int32),
            pltpu.VMEM((CHUNK, 128), _f32),
            pltpu.VMEM_SHARED((NPAD, 128), _f32),
        ],
    )
    def deg_kernel(dst_hbm, zeros_hbm, ones_hbm, out_hbm, dst_v, ones_v, acc_sh):
        c = lax.axis_index("c")
        s = lax.axis_index("s")
        g = c * NS + s
        r0 = s * RPT
        pltpu.sync_copy(ones_hbm, ones_v)
        pltpu.sync_copy(zeros_hbm.at[pl.ds(r0, RPT)], acc_sh.at[pl.ds(r0, RPT)])
        plsc.subcore_barrier()

        def body(j, carry):
            base = g * EPT + j * CHUNK
            pltpu.sync_copy(dst_hbm.at[pl.ds(base, CHUNK)], dst_v)
            pltpu.sync_copy(ones_v, acc_sh.at[dst_v], add=True)
            return carry

        lax.fori_loop(0, NCH, body, 0)
        plsc.subcore_barrier()
        pltpu.sync_copy(acc_sh.at[pl.ds(r0, RPT)], out_hbm.at[c, pl.ds(r0, RPT)])

    return deg_kernel


@functools.cache
def _make_prop(D):
    mesh = plsc.VectorSubcoreMesh(core_axis_name="c", subcore_axis_name="s")

    @functools.partial(
        pl.kernel, mesh=mesh,
        out_type=jax.ShapeDtypeStruct((NC, NPAD, D), _f32),
        scratch_types=[
            pltpu.VMEM((CHUNK,), jnp.int32),
            pltpu.VMEM((CHUNK,), jnp.int32),
            pltpu.VMEM((CHUNK, D), _f32),
            pltpu.VMEM_SHARED((NPAD, D), _f32),
            pltpu.SemaphoreType.DMA,
        ],
    )
    def prop(src_hbm, dst_hbm, table_hbm, zeros_hbm, out_hbm,
             src_v, dst_v, rows_v, acc_sh, sem):
        c = lax.axis_index("c")
        s = lax.axis_index("s")
        g = c * NS + s
        r0 = s * RPT
        pltpu.sync_copy(zeros_hbm.at[pl.ds(r0, RPT)], acc_sh.at[pl.ds(r0, RPT)])
        plsc.subcore_barrier()

        def body(j, carry):
            base = g * EPT + j * CHUNK
            pltpu.sync_copy(src_hbm.at[pl.ds(base, CHUNK)], src_v)
            pltpu.sync_copy(dst_hbm.at[pl.ds(base, CHUNK)], dst_v)
            pltpu.async_copy(table_hbm.at[src_v], rows_v, sem).wait()
            pltpu.sync_copy(rows_v, acc_sh.at[dst_v], add=True)
            return carry

        lax.fori_loop(0, NCH, body, 0)
        plsc.subcore_barrier()
        pltpu.sync_copy(acc_sh.at[pl.ds(r0, RPT)], out_hbm.at[c, pl.ds(r0, RPT)])

    return prop


def _dinv_from(deg_ref):
    dv = deg_ref[...]
    return lax.rsqrt(dv[0, :, :1] + dv[1, :, :1] + 1.0)


def _mm1_body(x_ref, w_ref, deg_ref, out_ref):
    out_ref[...] = jnp.dot(x_ref[...], w_ref[...],
                           preferred_element_type=_f32) * _dinv_from(deg_ref)


def _fuse2_body(acc_ref, t1_ref, deg_ref, b1_ref, w2_ref, out_ref):
    dinv = _dinv_from(deg_ref)
    a = acc_ref[...]
    z = (a[0] + a[1] + t1_ref[...]) * dinv + b1_ref[...]
    z = jnp.maximum(z, 0.0)
    out_ref[...] = jnp.dot(z, w2_ref[...], preferred_element_type=_f32) * dinv


def _fuse3_body(acc_ref, t2_ref, deg_ref, b2_ref, out_ref):
    dinv = _dinv_from(deg_ref)
    a = acc_ref[...]
    z = (a[0] + a[1] + t2_ref[...]) * dinv + b2_ref[...]
    sm = z[:, :D_OUT]
    m = jnp.max(sm, axis=1, keepdims=True)
    e = jnp.exp(sm - m)
    out_ref[...] = e / jnp.sum(e, axis=1, keepdims=True)


def _mm1(x, w1, deg_raw):
    return pl.pallas_call(
        _mm1_body,
        grid=(NBLK,),
        in_specs=[
            pl.BlockSpec((BM, D_IN), lambda i: (i, 0)),
            pl.BlockSpec((D_IN, D_HID), lambda i: (0, 0)),
            pl.BlockSpec((NC, BM, 128), lambda i: (0, i, 0)),
        ],
        out_specs=pl.BlockSpec((BM, D_HID), lambda i: (i, 0)),
        out_shape=jax.ShapeDtypeStruct((N, D_HID), _f32),
    )(x, w1, deg_raw)


def _fuse2(agg1, t1, deg_raw, b1, w2pad):
    return pl.pallas_call(
        _fuse2_body,
        grid=(NBLK,),
        in_specs=[
            pl.BlockSpec((NC, BM, D_HID), lambda i: (0, i, 0)),
            pl.BlockSpec((BM, D_HID), lambda i: (i, 0)),
            pl.BlockSpec((NC, BM, 128), lambda i: (0, i, 0)),
            pl.BlockSpec((1, D_HID), lambda i: (0, 0)),
            pl.BlockSpec((D_HID, D_PAD), lambda i: (0, 0)),
        ],
        out_specs=pl.BlockSpec((BM, D_PAD), lambda i: (i, 0)),
        out_shape=jax.ShapeDtypeStruct((N, D_PAD), _f32),
    )(agg1, t1, deg_raw, b1, w2pad)


def _fuse3(agg2, t2, deg_raw, b2):
    return pl.pallas_call(
        _fuse3_body,
        grid=(NBLK,),
        in_specs=[
            pl.BlockSpec((NC, BM, D_PAD), lambda i: (0, i, 0)),
            pl.BlockSpec((BM, D_PAD), lambda i: (i, 0)),
            pl.BlockSpec((NC, BM, 128), lambda i: (0, i, 0)),
            pl.BlockSpec((1, D_PAD), lambda i: (0, 0)),
        ],
        out_specs=pl.BlockSpec((BM, D_OUT), lambda i: (i, 0)),
        out_shape=jax.ShapeDtypeStruct((N, D_OUT), _f32),
    )(agg2, t2, deg_raw, b2)


def kernel(inputs, edges, W1, b1, W2, b2):
    src = edges[0]
    dst = edges[1]
    pad = EPAD - E
    src_pad = jnp.concatenate([src, jnp.zeros((pad,), jnp.int32)])
    junk = N + jnp.arange(pad, dtype=jnp.int32) % (NPAD - N)
    dst_pad = jnp.concatenate([dst, junk])

    zeros128 = jnp.zeros((NPAD, 128), _f32)
    ones128 = jnp.ones((CHUNK, 128), _f32)
    w2pad = jnp.pad(W2, ((0, 0), (0, D_PAD - D_OUT)))
    b2pad = jnp.pad(b2, (0, D_PAD - D_OUT)).reshape(1, D_PAD)

    deg_raw = _deg_kernel_fn()(dst_pad, zeros128, ones128)
    t1 = _mm1(inputs, W1, deg_raw)
    agg1 = _make_prop(D_HID)(src_pad, dst_pad, t1, zeros128)
    t2 = _fuse2(agg1, t1, deg_raw, b1.reshape(1, D_HID), w2pad)
    agg2 = _make_prop(D_PAD)(src_pad, dst_pad, t2, zeros128)
    return _fuse3(agg2, t2, deg_raw, b2pad)
